# baseline (speedup 1.0000x reference)
import functools

import jax
import jax.numpy as jnp
from jax.experimental import pallas as pl
from jax.experimental.pallas import tpu as pltpu


def _epilogue(logits, b2, k, num_experts):
    logits = logits + b2
    mx = jnp.max(logits, axis=1, keepdims=True)
    sh = logits - mx
    e = jnp.exp(sh)
    s = jnp.sum(e, axis=1, keepdims=True)

    t = jnp.sum(e * sh, axis=1, keepdims=True)
    ent_blk = jnp.sum(jnp.log(s) - t / s)

    bm = e.shape[0]
    lane = jax.lax.broadcasted_iota(jnp.int32, (bm, num_experts), 1)
    pbits = jax.lax.bitcast_convert_type(e, jnp.int32)
    key = jnp.bitwise_or(jnp.bitwise_and(pbits, jnp.int32(~63)),
                         (num_experts - 1) - lane)
    kms = []
    for _ in range(k):
        km = jnp.max(key, axis=1, keepdims=True)
        kms.append(km)
        key = jnp.where(key == km, jnp.int32(-1), key)
    km8 = jnp.concatenate(kms, axis=1)
    idx = (num_experts - 1) - jnp.bitwise_and(km8, jnp.int32(63))
    w = jax.lax.bitcast_convert_type(
        jnp.bitwise_and(km8, jnp.int32(~63)), jnp.float32)
    wts = w / jnp.sum(w, axis=1, keepdims=True)
    return wts, idx, ent_blk


def _main_body(x_ref, w1_ref, b1_ref, w2_ref, b2_ref,
               wts_ref, idx_ref, ent_ref, logits_ref,
               ent_acc_ref, *, k, num_experts):
    m = pl.program_id(0)

    wts, idx, ent_blk = _epilogue(logits_ref[...], b2_ref[...],
                                  k, num_experts)
    wts_ref[...] = wts
    idx_ref[...] = idx

    prev_acc = jnp.where(m == 0, 0.0, ent_acc_ref[0])
    new_acc = prev_acc + jnp.where(m == 0, 0.0, ent_blk)
    ent_acc_ref[0] = new_acc
    ent_ref[0] = new_acc

    hidden = jnp.dot(x_ref[...], w1_ref[...],
                     preferred_element_type=jnp.float32)
    hidden = jnp.maximum(hidden + b1_ref[...], 0.0)
    logits_ref[...] = jnp.dot(hidden, w2_ref[...],
                              preferred_element_type=jnp.float32)


def _last_body(logits_ref, b2_ref, ent_in_ref,
               wts_ref, idx_ref, ent_ref, *, k, num_experts):
    wts, idx, ent_blk = _epilogue(logits_ref[...], b2_ref[...],
                                  k, num_experts)
    wts_ref[...] = wts
    idx_ref[...] = idx
    ent_ref[0] = ent_in_ref[0] + ent_blk


def kernel(x, W1, b1, W2, b2):
    tokens, in_dim = x.shape
    hidden_dim, num_experts = W2.shape
    k = 8
    bm = 512
    num_m = tokens // bm

    b1r = b1.reshape(1, hidden_dim)
    b2r = b2.reshape(1, num_experts)

    main = functools.partial(_main_body, k=k, num_experts=num_experts)
    lastb = functools.partial(_last_body, k=k, num_experts=num_experts)

    wts_head, idx_head, ent_part, logits_last = pl.pallas_call(
        main,
        grid=(num_m,),
        in_specs=[
            pl.BlockSpec((bm, in_dim), lambda m: (m, 0)),
            pl.BlockSpec((in_dim, hidden_dim), lambda m: (0, 0)),
            pl.BlockSpec((1, hidden_dim), lambda m: (0, 0)),
            pl.BlockSpec((hidden_dim, num_experts), lambda m: (0, 0)),
            pl.BlockSpec((1, num_experts), lambda m: (0, 0)),
        ],
        out_specs=[
            pl.BlockSpec((bm, k), lambda m: (jnp.maximum(m - 1, 0), 0)),
            pl.BlockSpec((bm, k), lambda m: (jnp.maximum(m - 1, 0), 0)),
            pl.BlockSpec(memory_space=pltpu.SMEM),
            pl.BlockSpec((bm, num_experts), lambda m: (0, 0)),
        ],
        out_shape=[
            jax.ShapeDtypeStruct((tokens - bm, k), jnp.float32),
            jax.ShapeDtypeStruct((tokens - bm, k), jnp.int32),
            jax.ShapeDtypeStruct((1,), jnp.float32),
            jax.ShapeDtypeStruct((bm, num_experts), jnp.float32),
        ],
        scratch_shapes=[
            pltpu.SMEM((1,), jnp.float32),
        ],
    )(x, W1, b1r, W2, b2r)

    wts_tail, idx_tail, ent_sum = pl.pallas_call(
        lastb,
        in_specs=[
            pl.BlockSpec((bm, num_experts), lambda: (0, 0)),
            pl.BlockSpec((1, num_experts), lambda: (0, 0)),
            pl.BlockSpec(memory_space=pltpu.SMEM),
        ],
        out_specs=[
            pl.BlockSpec((bm, k), lambda: (0, 0)),
            pl.BlockSpec((bm, k), lambda: (0, 0)),
            pl.BlockSpec(memory_space=pltpu.SMEM),
        ],
        out_shape=[
            jax.ShapeDtypeStruct((bm, k), jnp.float32),
            jax.ShapeDtypeStruct((bm, k), jnp.int32),
            jax.ShapeDtypeStruct((1,), jnp.float32),
        ],
    )(logits_last, b2r, ent_part)

    wts = jnp.concatenate([wts_head, wts_tail], axis=0)
    idx = jnp.concatenate([idx_head, idx_tail], axis=0)
    uncertainty = (ent_sum[0] / tokens) / jnp.log(jnp.float32(num_experts))
    return wts, idx, uncertainty
